# Initial kernel scaffold; baseline (speedup 1.0000x reference)
import jax
import jax.numpy as jnp
from jax.experimental import pallas as pl


def kernel(x, edge_index, W_self1, W_neigh1, W_self2, W_neigh2):
    raise NotImplementedError("write your pallas kernel here")


# speedup vs baseline: 3.5198x; 3.5198x over previous
import functools

import jax
import jax.numpy as jnp
from jax import lax
from jax.experimental import pallas as pl
from jax.experimental.pallas import tpu as pltpu
from jax.experimental.pallas import tpu_sc as plsc

N = 10000
NPAD = 10240
E = 320000
D = 128
DH = D // 2

NC = 2
NS = 16
EPW = E // NS
CHUNK = 40
NCHUNK = EPW // CHUNK
ROWS_PER_TILE = NPAD // NS

_mesh = plsc.VectorSubcoreMesh(core_axis_name="c", subcore_axis_name="s")


def _sc_agg_body(src_hbm, dst_hbm, xh_hbm, agg_out, deg_out,
                 src_v, dst_v, buf, ones_buf, zbuf, zbuf16, acc_sh, deg_sh,
                 sem):
    cid = lax.axis_index("c")
    sid = lax.axis_index("s")

    def _zrow(r, _):
        for k in range(DH // 16):
            zbuf[r, pl.ds(k * 16, 16)] = jnp.zeros((16,), jnp.float32)
        zbuf16[r, :] = jnp.zeros((16,), jnp.float32)
        return 0
    lax.fori_loop(0, 64, _zrow, 0)

    def _orow(r, _):
        ones_buf[r, :] = jnp.ones((16,), jnp.float32)
        return 0
    lax.fori_loop(0, CHUNK, _orow, 0)

    def _zacc(b, _):
        pltpu.sync_copy(zbuf, acc_sh.at[pl.ds(sid * ROWS_PER_TILE + b * 64, 64)])
        pltpu.sync_copy(zbuf16, deg_sh.at[pl.ds(sid * ROWS_PER_TILE + b * 64, 64)])
        return 0
    lax.fori_loop(0, ROWS_PER_TILE // 64, _zacc, 0)

    plsc.subcore_barrier()

    pltpu.sync_copy(src_hbm.at[sid], src_v)
    pltpu.sync_copy(dst_hbm.at[sid], dst_v)

    def _edge_chunk(j, _):
        pltpu.async_copy(xh_hbm.at[cid].at[src_v.at[j]], buf, sem).wait()
        pltpu.sync_copy(buf, acc_sh.at[dst_v.at[j]], add=True)
        return 0
    lax.fori_loop(0, NCHUNK, _edge_chunk, 0)

    @pl.when(cid == 0)
    def _():
        def _deg_chunk(j, _):
            pltpu.sync_copy(ones_buf, deg_sh.at[dst_v.at[j]], add=True)
            return 0
        lax.fori_loop(0, NCHUNK, _deg_chunk, 0)

    plsc.subcore_barrier()

    pltpu.sync_copy(acc_sh.at[pl.ds(sid * ROWS_PER_TILE, ROWS_PER_TILE)],
                    agg_out.at[cid, pl.ds(sid * ROWS_PER_TILE, ROWS_PER_TILE)])

    @pl.when(cid == 0)
    def _():
        pltpu.sync_copy(deg_sh.at[pl.ds(sid * ROWS_PER_TILE, ROWS_PER_TILE)],
                        deg_out.at[pl.ds(sid * ROWS_PER_TILE, ROWS_PER_TILE)])


_sc_agg = pl.kernel(
    _sc_agg_body,
    out_type=[
        jax.ShapeDtypeStruct((NC, NPAD, DH), jnp.float32),
        jax.ShapeDtypeStruct((NPAD, 16), jnp.float32),
    ],
    mesh=_mesh,
    scratch_types=[
        pltpu.VMEM((NCHUNK, CHUNK), jnp.int32),
        pltpu.VMEM((NCHUNK, CHUNK), jnp.int32),
        pltpu.VMEM((CHUNK, DH), jnp.float32),
        pltpu.VMEM((CHUNK, 16), jnp.float32),
        pltpu.VMEM((64, DH), jnp.float32),
        pltpu.VMEM((64, 16), jnp.float32),
        pltpu.VMEM_SHARED((NPAD, DH), jnp.float32),
        pltpu.VMEM_SHARED((NPAD, 16), jnp.float32),
        pltpu.SemaphoreType.DMA,
    ],
    compiler_params=pltpu.CompilerParams(use_tc_tiling_on_sc=False),
)


BLK = 400
GRID = N // BLK


def _tc_layer_body(relu, x_ref, a0_ref, a1_ref, deg_ref, ws_ref,
                   wn_ref, out_ref, f_ref=None):
    deg16 = jnp.sum(deg_ref[...], axis=1, keepdims=True)
    inv = 16.0 / jnp.maximum(deg16, 16.0)
    mean0 = a0_ref[...] * inv
    mean1 = a1_ref[...] * inv
    wn = wn_ref[...]
    out = (jnp.dot(x_ref[...], ws_ref[...], preferred_element_type=jnp.float32)
           + jnp.dot(mean0, wn[:DH, :], preferred_element_type=jnp.float32)
           + jnp.dot(mean1, wn[DH:, :], preferred_element_type=jnp.float32))
    if relu:
        out = jnp.maximum(out, 0.0)
    out_ref[...] = out
    if f_ref is not None:
        fm = jnp.max(out, axis=0, keepdims=True)

        @pl.when(pl.program_id(0) == 0)
        def _():
            f_ref[...] = fm

        @pl.when(pl.program_id(0) > 0)
        def _():
            f_ref[...] = jnp.maximum(f_ref[...], fm)


_in_specs = [
    pl.BlockSpec((BLK, D), lambda i: (i, 0)),
    pl.BlockSpec((BLK, DH), lambda i: (i, 0)),
    pl.BlockSpec((BLK, DH), lambda i: (i, 0)),
    pl.BlockSpec((BLK, 16), lambda i: (i, 0)),
    pl.BlockSpec((D, D), lambda i: (0, 0)),
    pl.BlockSpec((D, D), lambda i: (0, 0)),
]

_tc_layer1 = pl.pallas_call(
    functools.partial(_tc_layer_body, True),
    grid=(GRID,),
    in_specs=_in_specs,
    out_specs=pl.BlockSpec((BLK, D), lambda i: (i, 0)),
    out_shape=jax.ShapeDtypeStruct((N, D), jnp.float32),
)

_tc_layer2 = pl.pallas_call(
    functools.partial(_tc_layer_body, False),
    grid=(GRID,),
    in_specs=_in_specs,
    out_specs=[
        pl.BlockSpec((BLK, D), lambda i: (i, 0)),
        pl.BlockSpec((1, D), lambda i: (0, 0)),
    ],
    out_shape=[
        jax.ShapeDtypeStruct((N, D), jnp.float32),
        jax.ShapeDtypeStruct((1, D), jnp.float32),
    ],
)


def _split_halves(x):
    return jnp.stack([x[:, :DH], x[:, DH:]])


def kernel(x, edge_index, W_self1, W_neigh1, W_self2, W_neigh2):
    src = edge_index[0].astype(jnp.int32).reshape(NS, NCHUNK, CHUNK)
    dst = edge_index[1].astype(jnp.int32).reshape(NS, NCHUNK, CHUNK)

    aggp, deg = _sc_agg(src, dst, _split_halves(x))
    h = _tc_layer1(x, aggp[0], aggp[1], deg, W_self1, W_neigh1)

    aggp2, _ = _sc_agg(src, dst, _split_halves(h))
    e, f = _tc_layer2(h, aggp2[0], aggp2[1], deg, W_self2, W_neigh2)
    return (f, e)


# speedup vs baseline: 8.2080x; 2.3320x over previous
import functools

import jax
import jax.numpy as jnp
from jax import lax
from jax.experimental import pallas as pl
from jax.experimental.pallas import tpu as pltpu
from jax.experimental.pallas import tpu_sc as plsc

N = 10000
NPAD = 10240
E = 320000
D = 128
DH = D // 2

NC = 2
NS = 16
EPW = E // NS
CHUNK = 80
NCHUNK = EPW // CHUNK
ROWS_PER_TILE = NPAD // NS

_mesh = plsc.VectorSubcoreMesh(core_axis_name="c", subcore_axis_name="s")


def _make_sc_agg(do_deg):
    def body(src_hbm, dst_hbm, xh_hbm, agg_out, *rest):
        if do_deg:
            (deg_out, src_v, dst_v, buf0, buf1, ones_buf, zbuf, zbuf16,
             acc_sh, deg_sh, sem0, sem1) = rest
        else:
            (src_v, dst_v, buf0, buf1, zbuf, acc_sh, sem0, sem1) = rest
        cid = lax.axis_index("c")
        sid = lax.axis_index("s")

        def _zrow(r, _):
            for k in range(DH // 16):
                zbuf[r, pl.ds(k * 16, 16)] = jnp.zeros((16,), jnp.float32)
            if do_deg:
                zbuf16[r, :] = jnp.zeros((16,), jnp.float32)
            return 0
        lax.fori_loop(0, 64, _zrow, 0)

        if do_deg:
            def _orow(r, _):
                ones_buf[r, :] = jnp.ones((16,), jnp.float32)
                return 0
            lax.fori_loop(0, CHUNK, _orow, 0)

        def _zacc(b, _):
            base = sid * ROWS_PER_TILE + b * 64
            pltpu.sync_copy(zbuf, acc_sh.at[pl.ds(base, 64)])
            if do_deg:
                pltpu.sync_copy(zbuf16, deg_sh.at[pl.ds(base, 64)])
            return 0
        lax.fori_loop(0, ROWS_PER_TILE // 64, _zacc, 0)

        plsc.subcore_barrier()

        pltpu.sync_copy(src_hbm.at[sid], src_v)
        pltpu.sync_copy(dst_hbm.at[sid], dst_v)

        xh = xh_hbm.at[cid]

        pltpu.async_copy(xh.at[src_v.at[0]], buf0, sem0)
        pltpu.async_copy(xh.at[src_v.at[1]], buf1, sem1)

        def _step(b, buf, sem, deg_core):
            pltpu.make_async_copy(xh.at[src_v.at[b]], buf, sem).wait()
            pltpu.sync_copy(buf, acc_sh.at[dst_v.at[b]], add=True)
            if do_deg:
                @pl.when(cid == deg_core)
                def _():
                    pltpu.sync_copy(ones_buf, deg_sh.at[dst_v.at[b]], add=True)

        def _pair(j, _):
            b0 = 2 * j
            _step(b0, buf0, sem0, 0)
            pltpu.async_copy(xh.at[src_v.at[b0 + 2]], buf0, sem0)
            _step(b0 + 1, buf1, sem1, 1)
            pltpu.async_copy(xh.at[src_v.at[b0 + 3]], buf1, sem1)
            return 0
        lax.fori_loop(0, NCHUNK // 2 - 1, _pair, 0)
        _step(NCHUNK - 2, buf0, sem0, 0)
        _step(NCHUNK - 1, buf1, sem1, 1)

        plsc.subcore_barrier()

        rows = pl.ds(sid * ROWS_PER_TILE, ROWS_PER_TILE)
        pltpu.sync_copy(acc_sh.at[rows], agg_out.at[cid, rows])
        if do_deg:
            pltpu.sync_copy(deg_sh.at[rows], deg_out.at[cid, rows])

    out_type = [jax.ShapeDtypeStruct((NC, NPAD, DH), jnp.float32)]
    scratch = [
        pltpu.VMEM((NCHUNK, CHUNK), jnp.int32),
        pltpu.VMEM((NCHUNK, CHUNK), jnp.int32),
        pltpu.VMEM((CHUNK, DH), jnp.float32),
        pltpu.VMEM((CHUNK, DH), jnp.float32),
        pltpu.VMEM((64, DH), jnp.float32),
        pltpu.VMEM_SHARED((NPAD, DH), jnp.float32),
        pltpu.SemaphoreType.DMA,
        pltpu.SemaphoreType.DMA,
    ]
    if do_deg:
        out_type.append(jax.ShapeDtypeStruct((NC, NPAD, 16), jnp.float32))
        scratch = (scratch[:4]
                   + [pltpu.VMEM((CHUNK, 16), jnp.float32),
                      scratch[4],
                      pltpu.VMEM((64, 16), jnp.float32)]
                   + [scratch[5],
                      pltpu.VMEM_SHARED((NPAD, 16), jnp.float32)]
                   + scratch[6:])
    return pl.kernel(
        body,
        out_type=out_type,
        mesh=_mesh,
        scratch_types=scratch,
        compiler_params=pltpu.CompilerParams(use_tc_tiling_on_sc=False),
    )


_sc_agg_deg = _make_sc_agg(True)
_sc_agg = _make_sc_agg(False)


BLK = 400
GRID = N // BLK


def _tc_layer_body(relu, x_ref, a0_ref, a1_ref, d0_ref, d1_ref, ws_ref,
                   wn_ref, out_ref, f_ref=None):
    deg16 = jnp.sum(d0_ref[...] + d1_ref[...], axis=1, keepdims=True)
    inv = 16.0 / jnp.maximum(deg16, 16.0)
    mean0 = a0_ref[...] * inv
    mean1 = a1_ref[...] * inv
    wn = wn_ref[...]
    out = (jnp.dot(x_ref[...], ws_ref[...], preferred_element_type=jnp.float32)
           + jnp.dot(mean0, wn[:DH, :], preferred_element_type=jnp.float32)
           + jnp.dot(mean1, wn[DH:, :], preferred_element_type=jnp.float32))
    if relu:
        out = jnp.maximum(out, 0.0)
    out_ref[...] = out
    if f_ref is not None:
        fm = jnp.max(out, axis=0, keepdims=True)

        @pl.when(pl.program_id(0) == 0)
        def _():
            f_ref[...] = fm

        @pl.when(pl.program_id(0) > 0)
        def _():
            f_ref[...] = jnp.maximum(f_ref[...], fm)


_in_specs = [
    pl.BlockSpec((BLK, D), lambda i: (i, 0)),
    pl.BlockSpec((BLK, DH), lambda i: (i, 0)),
    pl.BlockSpec((BLK, DH), lambda i: (i, 0)),
    pl.BlockSpec((BLK, 16), lambda i: (i, 0)),
    pl.BlockSpec((BLK, 16), lambda i: (i, 0)),
    pl.BlockSpec((D, D), lambda i: (0, 0)),
    pl.BlockSpec((D, D), lambda i: (0, 0)),
]

_tc_layer1 = pl.pallas_call(
    functools.partial(_tc_layer_body, True),
    grid=(GRID,),
    in_specs=_in_specs,
    out_specs=pl.BlockSpec((BLK, D), lambda i: (i, 0)),
    out_shape=jax.ShapeDtypeStruct((N, D), jnp.float32),
)

_tc_layer2 = pl.pallas_call(
    functools.partial(_tc_layer_body, False),
    grid=(GRID,),
    in_specs=_in_specs,
    out_specs=[
        pl.BlockSpec((BLK, D), lambda i: (i, 0)),
        pl.BlockSpec((1, D), lambda i: (0, 0)),
    ],
    out_shape=[
        jax.ShapeDtypeStruct((N, D), jnp.float32),
        jax.ShapeDtypeStruct((1, D), jnp.float32),
    ],
)


def _split_halves(x):
    return jnp.stack([x[:, :DH], x[:, DH:]])


def kernel(x, edge_index, W_self1, W_neigh1, W_self2, W_neigh2):
    src = edge_index[0].astype(jnp.int32).reshape(NS, NCHUNK, CHUNK)
    dst = edge_index[1].astype(jnp.int32).reshape(NS, NCHUNK, CHUNK)

    aggp, degp = _sc_agg_deg(src, dst, _split_halves(x))
    h = _tc_layer1(x, aggp[0], aggp[1], degp[0], degp[1], W_self1, W_neigh1)

    (aggp2,) = _sc_agg(src, dst, _split_halves(h))
    e, f = _tc_layer2(h, aggp2[0], aggp2[1], degp[0], degp[1],
                      W_self2, W_neigh2)
    return (f, e)


# speedup vs baseline: 11.2544x; 1.3711x over previous
import functools

import jax
import jax.numpy as jnp
from jax import lax
from jax.experimental import pallas as pl
from jax.experimental.pallas import tpu as pltpu
from jax.experimental.pallas import tpu_sc as plsc

N = 10000
NPAD = 10240
E = 320000
D = 128
DH = D // 2

NC = 2
NS = 16
EPW = E // NS
CHUNK = 80
NCHUNK = EPW // CHUNK
NBUF = 5
ROWS_PER_TILE = NPAD // NS

_mesh = plsc.VectorSubcoreMesh(core_axis_name="c", subcore_axis_name="s")


def _make_sc_agg(do_deg):
    def body(src_hbm, dst_hbm, xh_hbm, agg_out, *rest):
        if do_deg:
            deg_out, src_v, dst_v, *rest = rest
            bufs, rest = list(rest[:NBUF]), rest[NBUF:]
            ones_buf, zbuf, zbuf16, acc_sh, deg_sh, *rest = rest
        else:
            src_v, dst_v, *rest = rest
            bufs, rest = list(rest[:NBUF]), rest[NBUF:]
            zbuf, acc_sh, *rest = rest
        gsems, ssems = list(rest[:NBUF]), list(rest[NBUF:])
        cid = lax.axis_index("c")
        sid = lax.axis_index("s")

        def _zrow(r, _):
            for k in range(DH // 16):
                zbuf[r, pl.ds(k * 16, 16)] = jnp.zeros((16,), jnp.float32)
            if do_deg:
                zbuf16[r, :] = jnp.zeros((16,), jnp.float32)
            return 0
        lax.fori_loop(0, 64, _zrow, 0)

        if do_deg:
            def _orow(r, _):
                ones_buf[r, :] = jnp.ones((16,), jnp.float32)
                return 0
            lax.fori_loop(0, CHUNK, _orow, 0)

        def _zacc(b, _):
            base = sid * ROWS_PER_TILE + b * 64
            pltpu.sync_copy(zbuf, acc_sh.at[pl.ds(base, 64)])
            if do_deg:
                pltpu.sync_copy(zbuf16, deg_sh.at[pl.ds(base, 64)])
            return 0
        lax.fori_loop(0, ROWS_PER_TILE // 64, _zacc, 0)

        plsc.subcore_barrier()

        pltpu.sync_copy(src_hbm.at[sid], src_v)
        pltpu.sync_copy(dst_hbm.at[sid], dst_v)

        xh = xh_hbm.at[cid]

        for k in range(NBUF):
            pltpu.async_copy(xh.at[src_v.at[k]], bufs[k], gsems[k])

        iters = NCHUNK // NBUF

        def _ring(j, _):
            for k in range(NBUF):
                b = j * NBUF + k
                pltpu.make_async_copy(xh.at[src_v.at[b]], bufs[k],
                                      gsems[k]).wait()
                pltpu.sync_copy(bufs[k], acc_sh.at[dst_v.at[b]], add=True)
                if do_deg:
                    @pl.when(cid == (b % 2))
                    def _():
                        pltpu.sync_copy(ones_buf, deg_sh.at[dst_v.at[b]],
                                        add=True)

                @pl.when(j < iters - 1)
                def _():
                    pltpu.async_copy(xh.at[src_v.at[b + NBUF]], bufs[k],
                                     gsems[k])
            return 0
        lax.fori_loop(0, iters, _ring, 0)

        plsc.subcore_barrier()

        rows = pl.ds(sid * ROWS_PER_TILE, ROWS_PER_TILE)
        pltpu.sync_copy(acc_sh.at[rows], agg_out.at[cid, rows])
        if do_deg:
            pltpu.sync_copy(deg_sh.at[rows], deg_out.at[cid, rows])

    out_type = [jax.ShapeDtypeStruct((NC, NPAD, DH), jnp.float32)]
    if do_deg:
        out_type.append(jax.ShapeDtypeStruct((NC, NPAD, 16), jnp.float32))
    scratch = [
        pltpu.VMEM((NCHUNK, CHUNK), jnp.int32),
        pltpu.VMEM((NCHUNK, CHUNK), jnp.int32),
    ]
    scratch += [pltpu.VMEM((CHUNK, DH), jnp.float32) for _ in range(NBUF)]
    if do_deg:
        scratch.append(pltpu.VMEM((CHUNK, 16), jnp.float32))
    scratch.append(pltpu.VMEM((64, DH), jnp.float32))
    if do_deg:
        scratch.append(pltpu.VMEM((64, 16), jnp.float32))
    scratch.append(pltpu.VMEM_SHARED((NPAD, DH), jnp.float32))
    if do_deg:
        scratch.append(pltpu.VMEM_SHARED((NPAD, 16), jnp.float32))
    scratch += [pltpu.SemaphoreType.DMA for _ in range(2 * NBUF)]
    return pl.kernel(
        body,
        out_type=out_type,
        mesh=_mesh,
        scratch_types=scratch,
        compiler_params=pltpu.CompilerParams(use_tc_tiling_on_sc=False),
    )


_sc_agg_deg = _make_sc_agg(True)
_sc_agg = _make_sc_agg(False)


BLK = 400
GRID = N // BLK


def _tc_layer_body(relu, x_ref, a0_ref, a1_ref, d0_ref, d1_ref, ws_ref,
                   wn_ref, out_ref, f_ref=None):
    deg16 = jnp.sum(d0_ref[...] + d1_ref[...], axis=1, keepdims=True)
    inv = 16.0 / jnp.maximum(deg16, 16.0)
    mean0 = a0_ref[...] * inv
    mean1 = a1_ref[...] * inv
    wn = wn_ref[...]
    out = (jnp.dot(x_ref[...], ws_ref[...], preferred_element_type=jnp.float32)
           + jnp.dot(mean0, wn[:DH, :], preferred_element_type=jnp.float32)
           + jnp.dot(mean1, wn[DH:, :], preferred_element_type=jnp.float32))
    if relu:
        out = jnp.maximum(out, 0.0)
    out_ref[...] = out
    if f_ref is not None:
        fm = jnp.max(out, axis=0, keepdims=True)

        @pl.when(pl.program_id(0) == 0)
        def _():
            f_ref[...] = fm

        @pl.when(pl.program_id(0) > 0)
        def _():
            f_ref[...] = jnp.maximum(f_ref[...], fm)


_in_specs = [
    pl.BlockSpec((BLK, D), lambda i: (i, 0)),
    pl.BlockSpec((BLK, DH), lambda i: (i, 0)),
    pl.BlockSpec((BLK, DH), lambda i: (i, 0)),
    pl.BlockSpec((BLK, 16), lambda i: (i, 0)),
    pl.BlockSpec((BLK, 16), lambda i: (i, 0)),
    pl.BlockSpec((D, D), lambda i: (0, 0)),
    pl.BlockSpec((D, D), lambda i: (0, 0)),
]

_tc_layer1 = pl.pallas_call(
    functools.partial(_tc_layer_body, True),
    grid=(GRID,),
    in_specs=_in_specs,
    out_specs=pl.BlockSpec((BLK, D), lambda i: (i, 0)),
    out_shape=jax.ShapeDtypeStruct((N, D), jnp.float32),
)

_tc_layer2 = pl.pallas_call(
    functools.partial(_tc_layer_body, False),
    grid=(GRID,),
    in_specs=_in_specs,
    out_specs=[
        pl.BlockSpec((BLK, D), lambda i: (i, 0)),
        pl.BlockSpec((1, D), lambda i: (0, 0)),
    ],
    out_shape=[
        jax.ShapeDtypeStruct((N, D), jnp.float32),
        jax.ShapeDtypeStruct((1, D), jnp.float32),
    ],
)


def _split_halves(x):
    return jnp.stack([x[:, :DH], x[:, DH:]])


def kernel(x, edge_index, W_self1, W_neigh1, W_self2, W_neigh2):
    src = edge_index[0].astype(jnp.int32).reshape(NS, NCHUNK, CHUNK)
    dst = edge_index[1].astype(jnp.int32).reshape(NS, NCHUNK, CHUNK)

    aggp, degp = _sc_agg_deg(src, dst, _split_halves(x))
    h = _tc_layer1(x, aggp[0], aggp[1], degp[0], degp[1], W_self1, W_neigh1)

    (aggp2,) = _sc_agg(src, dst, _split_halves(h))
    e, f = _tc_layer2(h, aggp2[0], aggp2[1], degp[0], degp[1],
                      W_self2, W_neigh2)
    return (f, e)


# speedup vs baseline: 12.4386x; 1.1052x over previous
import functools

import jax
import jax.numpy as jnp
from jax import lax
from jax.experimental import pallas as pl
from jax.experimental.pallas import tpu as pltpu
from jax.experimental.pallas import tpu_sc as plsc

N = 10000
NPAD = 10240
E = 320000
D = 128
DH = D // 2

NC = 2
NS = 16
EPW = E // NS
CHUNK = 80
NCHUNK = EPW // CHUNK
NBUF = 5
ROWS_PER_TILE = NPAD // NS

_mesh = plsc.VectorSubcoreMesh(core_axis_name="c", subcore_axis_name="s")


def _make_sc_agg(do_deg):
    def body(src_hbm, dst_hbm, x_hbm, agg_out, *rest):
        if do_deg:
            deg_out, src_v, dst_v, *rest = rest
            bufs, rest = list(rest[:NBUF]), rest[NBUF:]
            ones_buf, zbuf, zbuf16, acc_sh, deg_sh, *rest = rest
        else:
            src_v, dst_v, *rest = rest
            bufs, rest = list(rest[:NBUF]), rest[NBUF:]
            zbuf, acc_sh, *rest = rest
        gsems = list(rest)
        cid = lax.axis_index("c")
        sid = lax.axis_index("s")

        def _zrow(r, _):
            for k in range(DH // 16):
                zbuf[r, pl.ds(k * 16, 16)] = jnp.zeros((16,), jnp.float32)
            if do_deg:
                zbuf16[r, :] = jnp.zeros((16,), jnp.float32)
            return 0
        lax.fori_loop(0, 64, _zrow, 0)

        if do_deg:
            def _orow(r, _):
                ones_buf[r, :] = jnp.ones((16,), jnp.float32)
                return 0
            lax.fori_loop(0, CHUNK, _orow, 0)

        def _zacc(b, _):
            base = sid * ROWS_PER_TILE + b * 64
            pltpu.sync_copy(zbuf, acc_sh.at[pl.ds(base, 64)])
            if do_deg:
                pltpu.sync_copy(zbuf16, deg_sh.at[pl.ds(base, 64)])
            return 0
        lax.fori_loop(0, ROWS_PER_TILE // 64, _zacc, 0)

        plsc.subcore_barrier()

        pltpu.sync_copy(src_hbm.at[sid], src_v)
        pltpu.sync_copy(dst_hbm.at[sid], dst_v)

        xh = x_hbm.at[cid]

        for k in range(NBUF):
            pltpu.async_copy(xh.at[src_v.at[k]], bufs[k], gsems[k])

        iters = NCHUNK // NBUF

        def _ring(j, _):
            for k in range(NBUF):
                b = j * NBUF + k
                pltpu.make_async_copy(xh.at[src_v.at[b]], bufs[k],
                                      gsems[k]).wait()
                pltpu.sync_copy(bufs[k], acc_sh.at[dst_v.at[b]], add=True)
                if do_deg:
                    @pl.when(cid == (b % 2))
                    def _():
                        pltpu.sync_copy(ones_buf, deg_sh.at[dst_v.at[b]],
                                        add=True)

                @pl.when(j < iters - 1)
                def _():
                    pltpu.async_copy(xh.at[src_v.at[b + NBUF]], bufs[k],
                                     gsems[k])
            return 0
        lax.fori_loop(0, iters, _ring, 0)

        plsc.subcore_barrier()

        rows = pl.ds(sid * ROWS_PER_TILE, ROWS_PER_TILE)
        pltpu.sync_copy(acc_sh.at[rows],
                        agg_out.at[rows, pl.ds(cid * DH, DH)])
        if do_deg:
            pltpu.sync_copy(deg_sh.at[rows], deg_out.at[cid, rows])

    out_type = [jax.ShapeDtypeStruct((NPAD, D), jnp.float32)]
    if do_deg:
        out_type.append(jax.ShapeDtypeStruct((NC, NPAD, 16), jnp.float32))
    scratch = [
        pltpu.VMEM((NCHUNK, CHUNK), jnp.int32),
        pltpu.VMEM((NCHUNK, CHUNK), jnp.int32),
    ]
    scratch += [pltpu.VMEM((CHUNK, DH), jnp.float32) for _ in range(NBUF)]
    if do_deg:
        scratch.append(pltpu.VMEM((CHUNK, 16), jnp.float32))
    scratch.append(pltpu.VMEM((64, DH), jnp.float32))
    if do_deg:
        scratch.append(pltpu.VMEM((64, 16), jnp.float32))
    scratch.append(pltpu.VMEM_SHARED((NPAD, DH), jnp.float32))
    if do_deg:
        scratch.append(pltpu.VMEM_SHARED((NPAD, 16), jnp.float32))
    scratch += [pltpu.SemaphoreType.DMA for _ in range(NBUF)]
    return pl.kernel(
        body,
        out_type=out_type,
        mesh=_mesh,
        scratch_types=scratch,
        compiler_params=pltpu.CompilerParams(use_tc_tiling_on_sc=False),
    )


_sc_agg_deg = _make_sc_agg(True)
_sc_agg = _make_sc_agg(False)


BLK = 400
GRID = N // BLK


def _tc_self_body(x_ref, w_ref, out_ref):
    out_ref[...] = jnp.dot(x_ref[...], w_ref[...],
                           preferred_element_type=jnp.float32)


_tc_self = pl.pallas_call(
    _tc_self_body,
    grid=(GRID,),
    in_specs=[
        pl.BlockSpec((BLK, D), lambda i: (i, 0)),
        pl.BlockSpec((D, D), lambda i: (0, 0)),
    ],
    out_specs=pl.BlockSpec((BLK, D), lambda i: (i, 0)),
    out_shape=jax.ShapeDtypeStruct((N, D), jnp.float32),
)


def _tc_combine_body(relu, xs_ref, agg_ref, d0_ref, d1_ref, wn_ref,
                     out_ref, f_ref=None):
    deg16 = jnp.sum(d0_ref[...] + d1_ref[...], axis=1, keepdims=True)
    inv = 16.0 / jnp.maximum(deg16, 16.0)
    mean = agg_ref[...] * inv
    out = xs_ref[...] + jnp.dot(mean, wn_ref[...],
                                preferred_element_type=jnp.float32)
    if relu:
        out = jnp.maximum(out, 0.0)
    out_ref[...] = out
    if f_ref is not None:
        fm = jnp.max(out, axis=0, keepdims=True)

        @pl.when(pl.program_id(0) == 0)
        def _():
            f_ref[...] = fm

        @pl.when(pl.program_id(0) > 0)
        def _():
            f_ref[...] = jnp.maximum(f_ref[...], fm)


_combine_in_specs = [
    pl.BlockSpec((BLK, D), lambda i: (i, 0)),
    pl.BlockSpec((BLK, D), lambda i: (i, 0)),
    pl.BlockSpec((BLK, 16), lambda i: (i, 0)),
    pl.BlockSpec((BLK, 16), lambda i: (i, 0)),
    pl.BlockSpec((D, D), lambda i: (0, 0)),
]

_tc_combine1 = pl.pallas_call(
    functools.partial(_tc_combine_body, True),
    grid=(GRID,),
    in_specs=_combine_in_specs,
    out_specs=pl.BlockSpec((BLK, D), lambda i: (i, 0)),
    out_shape=jax.ShapeDtypeStruct((N, D), jnp.float32),
)

_tc_combine2 = pl.pallas_call(
    functools.partial(_tc_combine_body, False),
    grid=(GRID,),
    in_specs=_combine_in_specs,
    out_specs=[
        pl.BlockSpec((BLK, D), lambda i: (i, 0)),
        pl.BlockSpec((1, D), lambda i: (0, 0)),
    ],
    out_shape=[
        jax.ShapeDtypeStruct((N, D), jnp.float32),
        jax.ShapeDtypeStruct((1, D), jnp.float32),
    ],
)


def _split_halves(x):
    return jnp.stack([x[:, :DH], x[:, DH:]])


def kernel(x, edge_index, W_self1, W_neigh1, W_self2, W_neigh2):
    src = edge_index[0].astype(jnp.int32).reshape(NS, NCHUNK, CHUNK)
    dst = edge_index[1].astype(jnp.int32).reshape(NS, NCHUNK, CHUNK)

    agg, degp = _sc_agg_deg(src, dst, _split_halves(x))
    xs = _tc_self(x, W_self1)
    h = _tc_combine1(xs, agg, degp[0], degp[1], W_neigh1)

    (agg2,) = _sc_agg(src, dst, _split_halves(h))
    hs = _tc_self(h, W_self2)
    e, f = _tc_combine2(hs, agg2, degp[0], degp[1], W_neigh2)
    return (f, e)


# speedup vs baseline: 14.2616x; 1.1466x over previous
import functools

import jax
import jax.numpy as jnp
from jax import lax
from jax.experimental import pallas as pl
from jax.experimental.pallas import tpu as pltpu
from jax.experimental.pallas import tpu_sc as plsc

N = 10000
NPAD = 10240
E = 320000
D = 128
DH = D // 2

NC = 2
NS = 16
EPW = E // NS
CHUNK = 80
NCHUNK = EPW // CHUNK
NBUF = 5
ROWS_PER_TILE = NPAD // NS

_mesh = plsc.VectorSubcoreMesh(core_axis_name="c", subcore_axis_name="s")


def _make_sc_agg(do_deg):
    def body(src_hbm, dst_hbm, x_hbm, agg_out, *rest):
        if do_deg:
            deg_out, src_v, dst_v, *rest = rest
            bufs, rest = list(rest[:NBUF]), rest[NBUF:]
            ones_buf, zbuf, zbuf16, acc_sh, deg_sh, *rest = rest
        else:
            src_v, dst_v, *rest = rest
            bufs, rest = list(rest[:NBUF]), rest[NBUF:]
            zbuf, acc_sh, *rest = rest
        gsems = list(rest)
        cid = lax.axis_index("c")
        sid = lax.axis_index("s")

        def _zrow(r, _):
            for k in range(DH // 16):
                zbuf[r, pl.ds(k * 16, 16)] = jnp.zeros((16,), jnp.float32)
            if do_deg:
                zbuf16[r, :] = jnp.zeros((16,), jnp.float32)
            return 0
        lax.fori_loop(0, 64, _zrow, 0)

        if do_deg:
            def _orow(r, _):
                ones_buf[r, :] = jnp.ones((16,), jnp.float32)
                return 0
            lax.fori_loop(0, CHUNK, _orow, 0)

        def _zacc(b, _):
            base = sid * ROWS_PER_TILE + b * 64
            pltpu.sync_copy(zbuf, acc_sh.at[pl.ds(base, 64)])
            if do_deg:
                pltpu.sync_copy(zbuf16, deg_sh.at[pl.ds(base, 64)])
            return 0
        lax.fori_loop(0, ROWS_PER_TILE // 64, _zacc, 0)

        plsc.subcore_barrier()

        pltpu.sync_copy(src_hbm.at[sid], src_v)
        pltpu.sync_copy(dst_hbm.at[sid], dst_v)

        def _xform(c):
            for k in range(CHUNK // 16):
                v = src_v[c, pl.ds(k * 16, 16)]
                src_v[c, pl.ds(k * 16, 16)] = v + v + cid

        def _xform_loop(c, _):
            _xform(c)
            return 0
        lax.fori_loop(0, NBUF, _xform_loop, 0)

        for k in range(NBUF):
            pltpu.async_copy(x_hbm.at[src_v.at[k]], bufs[k], gsems[k])

        iters = NCHUNK // NBUF

        def _ring(j, _):
            for k in range(NBUF):
                b = j * NBUF + k
                pltpu.make_async_copy(x_hbm.at[src_v.at[b]], bufs[k],
                                      gsems[k]).wait()
                pltpu.sync_copy(bufs[k], acc_sh.at[dst_v.at[b]], add=True)
                if do_deg:
                    @pl.when(cid == (b % 2))
                    def _():
                        pltpu.sync_copy(ones_buf, deg_sh.at[dst_v.at[b]],
                                        add=True)

                @pl.when(j < iters - 1)
                def _():
                    _xform(b + NBUF)
                    pltpu.async_copy(x_hbm.at[src_v.at[b + NBUF]], bufs[k],
                                     gsems[k])
            return 0
        lax.fori_loop(0, iters, _ring, 0)

        plsc.subcore_barrier()

        rows = pl.ds(sid * ROWS_PER_TILE, ROWS_PER_TILE)
        pltpu.sync_copy(acc_sh.at[rows],
                        agg_out.at[rows, pl.ds(cid * DH, DH)])
        if do_deg:
            pltpu.sync_copy(deg_sh.at[rows], deg_out.at[cid, rows])

    out_type = [jax.ShapeDtypeStruct((NPAD, D), jnp.float32)]
    if do_deg:
        out_type.append(jax.ShapeDtypeStruct((NC, NPAD, 16), jnp.float32))
    scratch = [
        pltpu.VMEM((NCHUNK, CHUNK), jnp.int32),
        pltpu.VMEM((NCHUNK, CHUNK), jnp.int32),
    ]
    scratch += [pltpu.VMEM((CHUNK, DH), jnp.float32) for _ in range(NBUF)]
    if do_deg:
        scratch.append(pltpu.VMEM((CHUNK, 16), jnp.float32))
    scratch.append(pltpu.VMEM((64, DH), jnp.float32))
    if do_deg:
        scratch.append(pltpu.VMEM((64, 16), jnp.float32))
    scratch.append(pltpu.VMEM_SHARED((NPAD, DH), jnp.float32))
    if do_deg:
        scratch.append(pltpu.VMEM_SHARED((NPAD, 16), jnp.float32))
    scratch += [pltpu.SemaphoreType.DMA for _ in range(NBUF)]
    return pl.kernel(
        body,
        out_type=out_type,
        mesh=_mesh,
        scratch_types=scratch,
        compiler_params=pltpu.CompilerParams(use_tc_tiling_on_sc=False),
    )


_sc_agg_deg = _make_sc_agg(True)
_sc_agg = _make_sc_agg(False)


BLK = 400
GRID = N // BLK


def _tc_self_body(x_ref, w_ref, out_ref):
    out_ref[...] = jnp.dot(x_ref[...], w_ref[...],
                           preferred_element_type=jnp.float32)


_tc_self = pl.pallas_call(
    _tc_self_body,
    grid=(GRID,),
    in_specs=[
        pl.BlockSpec((BLK, D), lambda i: (i, 0)),
        pl.BlockSpec((D, D), lambda i: (0, 0)),
    ],
    out_specs=pl.BlockSpec((BLK, D), lambda i: (i, 0)),
    out_shape=jax.ShapeDtypeStruct((N, D), jnp.float32),
)


def _tc_combine_body(relu, xs_ref, agg_ref, d0_ref, d1_ref, wn_ref,
                     out_ref, f_ref=None):
    deg16 = jnp.sum(d0_ref[0] + d1_ref[0], axis=1, keepdims=True)
    inv = 16.0 / jnp.maximum(deg16, 16.0)
    mean = agg_ref[...] * inv
    out = xs_ref[...] + jnp.dot(mean, wn_ref[...],
                                preferred_element_type=jnp.float32)
    if relu:
        out = jnp.maximum(out, 0.0)
    out_ref[...] = out
    if f_ref is not None:
        fm = jnp.max(out, axis=0, keepdims=True)

        @pl.when(pl.program_id(0) == 0)
        def _():
            f_ref[...] = fm

        @pl.when(pl.program_id(0) > 0)
        def _():
            f_ref[...] = jnp.maximum(f_ref[...], fm)


_combine_in_specs = [
    pl.BlockSpec((BLK, D), lambda i: (i, 0)),
    pl.BlockSpec((BLK, D), lambda i: (i, 0)),
    pl.BlockSpec((1, BLK, 16), lambda i: (0, i, 0)),
    pl.BlockSpec((1, BLK, 16), lambda i: (1, i, 0)),
    pl.BlockSpec((D, D), lambda i: (0, 0)),
]

_tc_combine1 = pl.pallas_call(
    functools.partial(_tc_combine_body, True),
    grid=(GRID,),
    in_specs=_combine_in_specs,
    out_specs=pl.BlockSpec((BLK, D), lambda i: (i, 0)),
    out_shape=jax.ShapeDtypeStruct((N, D), jnp.float32),
)

_tc_combine2 = pl.pallas_call(
    functools.partial(_tc_combine_body, False),
    grid=(GRID,),
    in_specs=_combine_in_specs,
    out_specs=[
        pl.BlockSpec((BLK, D), lambda i: (i, 0)),
        pl.BlockSpec((1, D), lambda i: (0, 0)),
    ],
    out_shape=[
        jax.ShapeDtypeStruct((N, D), jnp.float32),
        jax.ShapeDtypeStruct((1, D), jnp.float32),
    ],
)


def kernel(x, edge_index, W_self1, W_neigh1, W_self2, W_neigh2):
    src = edge_index[0].astype(jnp.int32).reshape(NS, NCHUNK, CHUNK)
    dst = edge_index[1].astype(jnp.int32).reshape(NS, NCHUNK, CHUNK)

    agg, degp = _sc_agg_deg(src, dst, x.reshape(2 * N, DH))
    xs = _tc_self(x, W_self1)
    h = _tc_combine1(xs, agg, degp, degp, W_neigh1)

    (agg2,) = _sc_agg(src, dst, h.reshape(2 * N, DH))
    hs = _tc_self(h, W_self2)
    e, f = _tc_combine2(hs, agg2, degp, degp, W_neigh2)
    return (f, e)


# speedup vs baseline: 15.5831x; 1.0927x over previous
import functools

import jax
import jax.numpy as jnp
from jax import lax
from jax.experimental import pallas as pl
from jax.experimental.pallas import tpu as pltpu
from jax.experimental.pallas import tpu_sc as plsc

N = 10000
NPAD = 10240
E = 320000
D = 128
DH = D // 2

NC = 2
NS = 16
EPW = E // NS
CHUNK = 80
NCHUNK = EPW // CHUNK
NBUF = 5
ROWS_PER_TILE = NPAD // NS

_mesh = plsc.VectorSubcoreMesh(core_axis_name="c", subcore_axis_name="s")


def _make_sc_agg(do_deg):
    def body(pk_hbm, x_hbm, agg_out, *rest):
        if do_deg:
            deg_out, src_v, dst_v, *rest = rest
            bufs, rest = list(rest[:NBUF]), rest[NBUF:]
            ones_buf, zbuf, zbuf16, acc_sh, deg_sh, *rest = rest
        else:
            src_v, dst_v, *rest = rest
            bufs, rest = list(rest[:NBUF]), rest[NBUF:]
            zbuf, acc_sh, *rest = rest
        gsems = list(rest)
        cid = lax.axis_index("c")
        sid = lax.axis_index("s")

        def _zrow(r, _):
            for k in range(DH // 16):
                zbuf[r, pl.ds(k * 16, 16)] = jnp.zeros((16,), jnp.float32)
            if do_deg:
                zbuf16[r, :] = jnp.zeros((16,), jnp.float32)
            return 0
        lax.fori_loop(0, 64, _zrow, 0)

        if do_deg:
            def _orow(r, _):
                ones_buf[r, :] = jnp.ones((16,), jnp.float32)
                return 0
            lax.fori_loop(0, CHUNK, _orow, 0)

        def _zacc(b, _):
            base = sid * ROWS_PER_TILE + b * 64
            pltpu.sync_copy(zbuf, acc_sh.at[pl.ds(base, 64)])
            if do_deg:
                pltpu.sync_copy(zbuf16, deg_sh.at[pl.ds(base, 64)])
            return 0
        lax.fori_loop(0, ROWS_PER_TILE // 64, _zacc, 0)

        plsc.subcore_barrier()

        pltpu.sync_copy(pk_hbm.at[sid], dst_v)

        def _xform(c):
            for k in range(CHUNK // 16):
                v = dst_v[c, pl.ds(k * 16, 16)]
                s = v & 0xFFFF
                src_v[c, pl.ds(k * 16, 16)] = s + s + cid
                dst_v[c, pl.ds(k * 16, 16)] = v >> 16

        def _xform_loop(c, _):
            _xform(c)
            return 0
        lax.fori_loop(0, NBUF, _xform_loop, 0)

        for k in range(NBUF):
            pltpu.async_copy(x_hbm.at[src_v.at[k]], bufs[k], gsems[k])

        iters = NCHUNK // NBUF

        def _ring(j, _):
            for k in range(NBUF):
                b = j * NBUF + k
                pltpu.make_async_copy(x_hbm.at[src_v.at[b]], bufs[k],
                                      gsems[k]).wait()
                pltpu.sync_copy(bufs[k], acc_sh.at[dst_v.at[b]], add=True)
                if do_deg:
                    @pl.when(cid == (b % 2))
                    def _():
                        pltpu.sync_copy(ones_buf, deg_sh.at[dst_v.at[b]],
                                        add=True)

                @pl.when(j < iters - 1)
                def _():
                    _xform(b + NBUF)
                    pltpu.async_copy(x_hbm.at[src_v.at[b + NBUF]], bufs[k],
                                     gsems[k])
            return 0
        lax.fori_loop(0, iters, _ring, 0)

        plsc.subcore_barrier()

        rows = pl.ds(sid * ROWS_PER_TILE, ROWS_PER_TILE)
        pltpu.sync_copy(acc_sh.at[rows],
                        agg_out.at[rows, pl.ds(cid * DH, DH)])
        if do_deg:
            pltpu.sync_copy(deg_sh.at[rows], deg_out.at[cid, rows])

    out_type = [jax.ShapeDtypeStruct((NPAD, D), jnp.float32)]
    if do_deg:
        out_type.append(jax.ShapeDtypeStruct((NC, NPAD, 16), jnp.float32))
    scratch = [
        pltpu.VMEM((NCHUNK, CHUNK), jnp.int32),
        pltpu.VMEM((NCHUNK, CHUNK), jnp.int32),
    ]
    scratch += [pltpu.VMEM((CHUNK, DH), jnp.float32) for _ in range(NBUF)]
    if do_deg:
        scratch.append(pltpu.VMEM((CHUNK, 16), jnp.float32))
    scratch.append(pltpu.VMEM((64, DH), jnp.float32))
    if do_deg:
        scratch.append(pltpu.VMEM((64, 16), jnp.float32))
    scratch.append(pltpu.VMEM_SHARED((NPAD, DH), jnp.float32))
    if do_deg:
        scratch.append(pltpu.VMEM_SHARED((NPAD, 16), jnp.float32))
    scratch += [pltpu.SemaphoreType.DMA for _ in range(NBUF)]
    return pl.kernel(
        body,
        out_type=out_type,
        mesh=_mesh,
        scratch_types=scratch,
        compiler_params=pltpu.CompilerParams(use_tc_tiling_on_sc=False),
    )


_sc_agg_deg = _make_sc_agg(True)
_sc_agg = _make_sc_agg(False)


BLK = 400
GRID = N // BLK
CBLK = 2000
CGRID = N // CBLK


def _tc_self_body(x_ref, w_ref, out_ref):
    out_ref[...] = jnp.dot(x_ref[...], w_ref[...],
                           preferred_element_type=jnp.float32)


_tc_self = pl.pallas_call(
    _tc_self_body,
    grid=(GRID,),
    in_specs=[
        pl.BlockSpec((BLK, D), lambda i: (i, 0)),
        pl.BlockSpec((D, D), lambda i: (0, 0)),
    ],
    out_specs=pl.BlockSpec((BLK, D), lambda i: (i, 0)),
    out_shape=jax.ShapeDtypeStruct((N, D), jnp.float32),
)


def _tc_combine_body(relu, xs_ref, agg_ref, d0_ref, d1_ref, wn_ref,
                     out_ref, f_ref=None):
    deg16 = jnp.sum(d0_ref[0] + d1_ref[0], axis=1, keepdims=True)
    inv = 16.0 / jnp.maximum(deg16, 16.0)
    mean = agg_ref[...] * inv
    out = xs_ref[...] + jnp.dot(mean, wn_ref[...],
                                preferred_element_type=jnp.float32)
    if relu:
        out = jnp.maximum(out, 0.0)
    out_ref[...] = out
    if f_ref is not None:
        fm = jnp.max(out, axis=0, keepdims=True)

        @pl.when(pl.program_id(0) == 0)
        def _():
            f_ref[...] = fm

        @pl.when(pl.program_id(0) > 0)
        def _():
            f_ref[...] = jnp.maximum(f_ref[...], fm)


_combine_in_specs = [
    pl.BlockSpec((CBLK, D), lambda i: (i, 0)),
    pl.BlockSpec((CBLK, D), lambda i: (i, 0)),
    pl.BlockSpec((1, CBLK, 16), lambda i: (0, i, 0)),
    pl.BlockSpec((1, CBLK, 16), lambda i: (1, i, 0)),
    pl.BlockSpec((D, D), lambda i: (0, 0)),
]

_tc_combine1 = pl.pallas_call(
    functools.partial(_tc_combine_body, True),
    grid=(CGRID,),
    in_specs=_combine_in_specs,
    out_specs=pl.BlockSpec((CBLK, D), lambda i: (i, 0)),
    out_shape=jax.ShapeDtypeStruct((N, D), jnp.float32),
)

_tc_combine2 = pl.pallas_call(
    functools.partial(_tc_combine_body, False),
    grid=(CGRID,),
    in_specs=_combine_in_specs,
    out_specs=[
        pl.BlockSpec((CBLK, D), lambda i: (i, 0)),
        pl.BlockSpec((1, D), lambda i: (0, 0)),
    ],
    out_shape=[
        jax.ShapeDtypeStruct((N, D), jnp.float32),
        jax.ShapeDtypeStruct((1, D), jnp.float32),
    ],
)


def kernel(x, edge_index, W_self1, W_neigh1, W_self2, W_neigh2):
    ei = edge_index.astype(jnp.int32)
    packed = (jnp.left_shift(ei[1], 16) | ei[0]).reshape(NS, NCHUNK, CHUNK)

    agg, degp = _sc_agg_deg(packed, x.reshape(2 * N, DH))
    xs = _tc_self(x, W_self1)
    h = _tc_combine1(xs, agg, degp, degp, W_neigh1)

    (agg2,) = _sc_agg(packed, h.reshape(2 * N, DH))
    hs = _tc_self(h, W_self2)
    e, f = _tc_combine2(hs, agg2, degp, degp, W_neigh2)
    return (f, e)
